# baseline (speedup 1.0000x reference)
import jax
import jax.numpy as jnp
from jax import lax
from jax.experimental import pallas as pl
from jax.experimental.pallas import tpu as pltpu

_TOPK_FRAC = 0.15


def _fused_body(x_ref, w1_ref, w2_ref, o_ref, keep_ref):
    xb = x_ref[0]
    c = xb.shape[1]
    top_k = float(int(c * _TOPK_FRAC))

    m = jnp.mean(xb, axis=0, keepdims=True)

    hdn = lax.dot_general(m, w1_ref[...], (((1,), (1,)), ((), ())))
    hdn = jnp.maximum(hdn, 0.0)
    z = lax.dot_general(hdn, w2_ref[...], (((1,), (1,)), ((), ())))
    s_row = jax.nn.sigmoid(z)
    s_col = jnp.transpose(s_row)

    gt = (s_col > s_row).astype(jnp.float32)
    cnt_gt = jnp.sum(gt, axis=0, keepdims=True)

    t = jnp.min(jnp.where(cnt_gt < top_k, s_row, 2.0), axis=1, keepdims=True)

    ge = s_row >= t
    cnt_ge = jnp.sum(ge.astype(jnp.float32))

    keep_ref[...] = 1.0 - ge.astype(jnp.float32)

    @pl.when(cnt_ge > top_k)
    def _tie():
        drop_gt = s_row > t
        eqc = s_col == t
        ltm = (
            lax.broadcasted_iota(jnp.int32, (c, c), 0)
            < lax.broadcasted_iota(jnp.int32, (c, c), 1)
        )
        cumex = jnp.sum(
            (eqc & ltm).astype(jnp.float32), axis=0, keepdims=True
        )
        need = top_k - jnp.sum(drop_gt.astype(jnp.float32))
        drop = drop_gt | ((s_row == t) & (cumex < need))
        keep_ref[...] = 1.0 - drop.astype(jnp.float32)

    o_ref[0] = xb * keep_ref[...]


def kernel(x, w1, w2):
    b, c, h, w = x.shape
    hw = h * w
    xt = jnp.transpose(x, (0, 2, 3, 1)).reshape(b, hw, c)
    out = pl.pallas_call(
        _fused_body,
        grid=(b,),
        in_specs=[
            pl.BlockSpec((1, hw, c), lambda i: (i, 0, 0)),
            pl.BlockSpec(w1.shape, lambda i: (0, 0)),
            pl.BlockSpec(w2.shape, lambda i: (0, 0)),
        ],
        out_specs=pl.BlockSpec((1, hw, c), lambda i: (i, 0, 0)),
        out_shape=jax.ShapeDtypeStruct((b, hw, c), x.dtype),
        scratch_shapes=[pltpu.VMEM((1, c), jnp.float32)],
        compiler_params=pltpu.CompilerParams(
            dimension_semantics=("parallel",),
        ),
    )(xt, w1, w2)
    return jnp.transpose(out.reshape(b, h, w, c), (0, 3, 1, 2))
